# Initial kernel scaffold; baseline (speedup 1.0000x reference)
import jax
import jax.numpy as jnp
from jax.experimental import pallas as pl


def kernel(x0, x1, emb_table, lstm_kernel, lstm_rec, lstm_bias, dense_W, dense_b):
    raise NotImplementedError("write your pallas kernel here")


# speedup vs baseline: 1.3517x; 1.3517x over previous
import functools

import jax
import jax.numpy as jnp
from jax import lax
from jax.experimental import pallas as pl
from jax.experimental.pallas import tpu as pltpu
from jax.experimental.pallas import tpu_sc as plsc

V = 100000
EMB = 400
U = 16
G4 = 4 * U
FEAT = 16
B = 1024
L = 200
LB = B * L


_S1_ROWS = 1000


def _proj_body(emb_ref, we_ref, out_ref):
    out_ref[...] = jnp.dot(emb_ref[...], we_ref[...],
                           preferred_element_type=jnp.float32)


def _project_table(emb_table, w_e):
    return pl.pallas_call(
        _proj_body,
        grid=(V // _S1_ROWS,),
        in_specs=[
            pl.BlockSpec((_S1_ROWS, EMB), lambda i: (i, 0)),
            pl.BlockSpec((EMB, G4), lambda i: (0, 0)),
        ],
        out_specs=pl.BlockSpec((_S1_ROWS, G4), lambda i: (i, 0)),
        out_shape=jax.ShapeDtypeStruct((V, G4), jnp.float32),
    )(emb_table, w_e)


_NC, _NS = 2, 16
_NW = _NC * _NS
_ROWS_W = LB // _NW
_JCH = 128
_JPS = 10
_SUP = _ROWS_W // (_JPS * _JCH)


def _sc_gather_body(table_hbm, idx_hbm, out_hbm, idx_v, rows_v, sem):
    wid = lax.axis_index("s") * _NC + lax.axis_index("c")
    row0 = wid * _ROWS_W
    pltpu.sync_copy(idx_hbm.at[wid], idx_v)

    @pl.loop(0, _SUP)
    def _super(s):
        copies = []
        for j in range(_JPS):
            copies.append(pltpu.async_copy(
                table_hbm.at[idx_v.at[s * _JPS + j]],
                rows_v.at[pl.ds(j * _JCH, _JCH)], sem))
        for c in copies:
            c.wait()
        pltpu.sync_copy(
            rows_v, out_hbm.at[pl.ds(row0 + s * _JPS * _JCH, _JPS * _JCH)])


@functools.cache
def _make_sc_gather():
    return functools.partial(
        pl.kernel,
        out_type=jax.ShapeDtypeStruct((LB, G4), jnp.float32),
        mesh=plsc.VectorSubcoreMesh(core_axis_name="c", subcore_axis_name="s"),
        scratch_types=[
            pltpu.VMEM((_ROWS_W // _JCH, _JCH), jnp.int32),
            pltpu.VMEM((_JPS * _JCH, G4), jnp.float32),
            pltpu.SemaphoreType.DMA,
        ],
        compiler_params=pltpu.CompilerParams(use_tc_tiling_on_sc=False),
    )(_sc_gather_body)


def _sc_gather(table, idx2d):
    return _make_sc_gather()(table, idx2d)


_TCH = 10


def _lstm_body(g_ref, x1_ref, wf_ref, rec_ref, bias_ref, h_out_ref,
               h_s, c_s):
    i = pl.program_id(0)

    @pl.when(i == 0)
    def _init():
        h_s[...] = jnp.zeros_like(h_s)
        c_s[...] = jnp.zeros_like(c_s)

    def step(t, hc):
        h, c = hc
        z = (g_ref[t]
             + jnp.dot(x1_ref[t], wf_ref[...],
                       preferred_element_type=jnp.float32)
             + jnp.dot(h, rec_ref[...], preferred_element_type=jnp.float32)
             + bias_ref[...])
        i_g = jax.nn.sigmoid(z[:, 0:U])
        f_g = jax.nn.sigmoid(z[:, U:2 * U])
        g_g = jnp.tanh(z[:, 2 * U:3 * U])
        o_g = jax.nn.sigmoid(z[:, 3 * U:4 * U])
        c_new = f_g * c + i_g * g_g
        h_new = o_g * jnp.tanh(c_new)
        return h_new, c_new

    h, c = lax.fori_loop(0, _TCH, step, (h_s[...], c_s[...]))
    h_s[...] = h
    c_s[...] = c
    h_out_ref[...] = h


def _lstm(g3, x1t, w_f, rec, bias2d):
    return pl.pallas_call(
        _lstm_body,
        grid=(L // _TCH,),
        in_specs=[
            pl.BlockSpec((_TCH, B, G4), lambda i: (i, 0, 0)),
            pl.BlockSpec((_TCH, B, FEAT), lambda i: (i, 0, 0)),
            pl.BlockSpec((FEAT, G4), lambda i: (0, 0)),
            pl.BlockSpec((U, G4), lambda i: (0, 0)),
            pl.BlockSpec((1, G4), lambda i: (0, 0)),
        ],
        out_specs=pl.BlockSpec((B, U), lambda i: (0, 0)),
        out_shape=jax.ShapeDtypeStruct((B, U), jnp.float32),
        scratch_shapes=[
            pltpu.VMEM((B, U), jnp.float32),
            pltpu.VMEM((B, U), jnp.float32),
        ],
    )(g3, x1t, w_f, rec, bias2d)


_BB = 32
_VCH = 2048
_VPAD = 100352
_NVC = _VPAD // _VCH
_TAIL = V - (_NVC - 1) * _VCH


def _head_body(h_ref, w_ref, b_ref, out_ref):
    hv = h_ref[...]
    total = jnp.zeros((_BB, 1), jnp.float32)
    for j in range(_NVC):
        lg = (jnp.dot(hv, w_ref[:, j * _VCH:(j + 1) * _VCH],
                      preferred_element_type=jnp.float32)
              + b_ref[:, j * _VCH:(j + 1) * _VCH])
        e = jnp.exp(lg)
        total = total + jnp.sum(e, axis=1, keepdims=True)
        if j < _NVC - 1:
            out_ref[:, j * _VCH:(j + 1) * _VCH] = e
        else:
            out_ref[:, j * _VCH:j * _VCH + _TAIL] = e[:, :_TAIL]
    inv = 1.0 / total
    for j in range(_NVC):
        if j < _NVC - 1:
            out_ref[:, j * _VCH:(j + 1) * _VCH] = (
                out_ref[:, j * _VCH:(j + 1) * _VCH] * inv)
        else:
            out_ref[:, j * _VCH:j * _VCH + _TAIL] = (
                out_ref[:, j * _VCH:j * _VCH + _TAIL] * inv)


def _softmax_head(h, w_pad, b_pad):
    return pl.pallas_call(
        _head_body,
        grid=(B // _BB,),
        in_specs=[
            pl.BlockSpec((_BB, U), lambda i: (i, 0)),
            pl.BlockSpec((U, _VPAD), lambda i: (0, 0)),
            pl.BlockSpec((1, _VPAD), lambda i: (0, 0)),
        ],
        out_specs=pl.BlockSpec((_BB, V), lambda i: (i, 0)),
        out_shape=jax.ShapeDtypeStruct((B, V), jnp.float32),
    )(h, w_pad, b_pad)


def kernel(x0, x1, emb_table, lstm_kernel, lstm_rec, lstm_bias, dense_W,
           dense_b):
    w_e = lstm_kernel[:EMB]
    w_f = lstm_kernel[EMB:]
    bias2d = lstm_bias.reshape(1, G4)

    proj = _project_table(emb_table, w_e)

    idx3d = x0.T.reshape(_NW, _ROWS_W // _JCH, _JCH)
    g = _sc_gather(proj, idx3d)
    g3 = g.reshape(L, B, G4)

    x1t = jnp.swapaxes(x1, 0, 1)
    h = _lstm(g3, x1t, w_f, lstm_rec, bias2d)

    w_pad = jnp.pad(dense_W, ((0, 0), (0, _VPAD - V)))
    b_pad = jnp.pad(dense_b.reshape(1, V), ((0, 0), (0, _VPAD - V)),
                    constant_values=-1e9)
    return _softmax_head(h, w_pad, b_pad)
